# baseline (speedup 1.0000x reference)
import functools

import jax
import jax.numpy as jnp
from jax import lax
from jax.experimental import pallas as pl
from jax.experimental.pallas import tpu as pltpu
from jax.experimental.pallas import tpu_sc as plsc

B = 16384
D = 64
NC = 2
NS = 16
L = 16
NW = NC * NS
BPW = B // NW
CH = 64
NCH = BPW // CH


def _sc_body(pixel_h, frame_h, wv_h, u_h, v_h, u1_h, u2_h, v1_h, v2_h,
             x_h,
             idxp, idxf, bu, bv, bu1, bu2, bv1, bv2, wv, xbuf, sem):
    wid = lax.axis_index("s") * NC + lax.axis_index("c")
    base = wid * BPW
    pltpu.sync_copy(wv_h, wv)
    iota = lax.iota(jnp.int32, L)
    zero = jnp.zeros((L,), jnp.float32)
    one = jnp.full((L,), 1.0, jnp.float32)
    b1v = wv[3 * D]
    w2v = wv[3 * D + 1]
    b2v = wv[3 * D + 2]
    for c in range(NCH):
        off = base + c * CH
        pltpu.sync_copy(pixel_h.at[pl.ds(off, CH)], idxp)
        pltpu.sync_copy(frame_h.at[pl.ds(off, CH)], idxf)
        cps = [
            pltpu.async_copy(u_h.at[idxp], bu, sem),
            pltpu.async_copy(v_h.at[idxf], bv, sem),
            pltpu.async_copy(u1_h.at[idxp], bu1, sem),
            pltpu.async_copy(u2_h.at[idxp], bu2, sem),
            pltpu.async_copy(v1_h.at[idxf], bv1, sem),
            pltpu.async_copy(v2_h.at[idxf], bv2, sem),
        ]
        for cp in cps:
            cp.wait()
        for g in range(CH // L):
            rowv = iota + g * L

            def body(d, acc):
                cols = jnp.full((L,), d, jnp.int32)
                u = plsc.load_gather(bu, [rowv, cols])
                v = plsc.load_gather(bv, [rowv, cols])
                u1 = plsc.load_gather(bu1, [rowv, cols])
                u2 = plsc.load_gather(bu2, [rowv, cols])
                v1 = plsc.load_gather(bv1, [rowv, cols])
                v2 = plsc.load_gather(bv2, [rowv, cols])
                wa = wv[d]
                wb = wv[D + d]
                wc = wv[2 * D + d]
                dp = (jnp.maximum(u1, zero) * jnp.maximum(v1, zero)
                      + jnp.maximum(u2, zero) * jnp.maximum(v2, zero))
                return acc + u * wa + v * wb + dp * wc

            acc = lax.fori_loop(0, D, body, jnp.zeros((L,), jnp.float32))
            h = jnp.maximum(acc + b1v, zero)
            t = h * w2v + b2v
            xg = one / (one + jnp.exp(-t))
            xbuf[pl.ds(g * L, L)] = xg
        pltpu.sync_copy(xbuf, x_h.at[pl.ds(off, CH)])


_sc_call = functools.partial(
    pl.kernel,
    out_type=jax.ShapeDtypeStruct((B,), jnp.float32),
    mesh=plsc.VectorSubcoreMesh(
        core_axis_name="c", subcore_axis_name="s", num_cores=NC,
        num_subcores=NS),
    scratch_types=[
        pltpu.VMEM((CH,), jnp.int32),
        pltpu.VMEM((CH,), jnp.int32),
        pltpu.VMEM((CH, D), jnp.float32),
        pltpu.VMEM((CH, D), jnp.float32),
        pltpu.VMEM((CH, D), jnp.float32),
        pltpu.VMEM((CH, D), jnp.float32),
        pltpu.VMEM((CH, D), jnp.float32),
        pltpu.VMEM((CH, D), jnp.float32),
        pltpu.VMEM((200, L), jnp.float32),
        pltpu.VMEM((CH,), jnp.float32),
        pltpu.SemaphoreType.DMA,
    ],
    compiler_params=pltpu.CompilerParams(
        needs_layout_passes=False, use_tc_tiling_on_sc=False),
)(_sc_body)


def _tail_body(x_ref, t_ref, s1, bs1, s2, bs2, s3, bs3, s4, bs4, o_ref):
    s = t_ref[...] - x_ref[...]
    h1 = [jnp.maximum(s * s1[0, k] + bs1[k], 0.0) for k in range(10)]
    h2 = [jnp.maximum(sum(h1[j] * s2[j, k] for j in range(10)) + bs2[k], 0.0)
          for k in range(10)]
    h3 = [jnp.maximum(sum(h2[j] * s3[j, k] for j in range(10)) + bs3[k], 0.0)
          for k in range(10)]
    o = sum(h3[j] * s4[j, 0] for j in range(10)) + bs4[0]
    o_ref[...] = 1.0 / (1.0 + jnp.exp(-o))


def _tail_call(x2d, t2d, S1, bs1, S2, bs2, S3, bs3, S4, bs4):
    smem = pl.BlockSpec(memory_space=pltpu.SMEM)
    return pl.pallas_call(
        _tail_body,
        out_shape=jax.ShapeDtypeStruct(x2d.shape, jnp.float32),
        in_specs=[pl.BlockSpec(memory_space=pltpu.VMEM),
                  pl.BlockSpec(memory_space=pltpu.VMEM),
                  smem, smem, smem, smem, smem, smem, smem, smem],
        out_specs=pl.BlockSpec(memory_space=pltpu.VMEM),
    )(x2d, t2d, S1, bs1, S2, bs2, S3, bs3, S4, bs4)


def kernel(pixel, frame, target, U, V, Up1, Up2, Vp1, Vp2, W1, b1, W2, b2,
           S1, bs1, S2, bs2, S3, bs3, S4, bs4):
    pixel_i = pixel.astype(jnp.int32)
    frame_i = frame.astype(jnp.int32)
    wflat = jnp.concatenate([
        W1.reshape(-1), b1.reshape(-1), W2.reshape(-1), b2.reshape(-1),
        jnp.zeros((200 - (3 * D + 3),), jnp.float32)])
    wv = jnp.broadcast_to(wflat[:, None], (200, L))
    x = _sc_call(pixel_i, frame_i, wv, U, V, Up1, Up2, Vp1, Vp2)
    x2d = x.reshape(128, 128)
    t2d = target.reshape(128, 128)
    s2d = _tail_call(x2d, t2d, S1, bs1, S2, bs2, S3, bs3, S4, bs4)
    return (x.reshape(B, 1), s2d.reshape(B, 1))
